# baseline (speedup 1.0000x reference)
import functools

import jax
import jax.numpy as jnp
from jax.experimental import pallas as pl
from jax.experimental.pallas import tpu as pltpu


def _fused_body(inv_hw, x_ref, s_ref, wc_ref, bc_ref, wp_ref, bp_ref,
                scores_ref, deltas_ref):
    xb = x_ref[...].astype(jnp.bfloat16)
    pooled = jax.lax.dot_general(
        xb, s_ref[...], (((1,), (0,)), ((), ())),
        preferred_element_type=jnp.float32) * inv_hw
    dn = (((1,), (1,)), ((), ()))
    scores_ref[...] = jax.lax.dot_general(
        pooled, wc_ref[...], dn,
        preferred_element_type=jnp.float32) + bc_ref[...]
    deltas_ref[...] = jax.lax.dot_general(
        pooled, wp_ref[...], dn,
        preferred_element_type=jnp.float32) + bp_ref[...]


def kernel(x, w_cls, b_cls, w_pred, b_pred):
    num_classes = w_cls.shape[0]
    nc4 = w_pred.shape[0]

    if x.ndim == 4:
        N, C, H, W = x.shape
        hw = H * W
    else:
        N, C = x.shape
        hw = 1
    chw = C * hw
    xflat = x.reshape(N, chw)

    w_cls = jnp.asarray(w_cls, jnp.float32)
    w_pred = jnp.asarray(w_pred, jnp.float32)
    bc2 = jnp.asarray(b_cls, jnp.float32).reshape(1, num_classes)
    bp2 = jnp.asarray(b_pred, jnp.float32).reshape(1, nc4)

    sel = jnp.repeat(jnp.eye(C, dtype=jnp.bfloat16), hw, axis=0)

    if N % 128 == 0:
        tn = 128
    elif N % 8 == 0:
        tn = 8
    else:
        tn = N
    grid = (N // tn,)

    itemsize = jnp.dtype(x.dtype).itemsize
    cost = pl.CostEstimate(
        flops=int(2 * N * chw * C + 2 * N * C * (num_classes + nc4)),
        transcendentals=0,
        bytes_accessed=int(N * chw * itemsize + sel.size * 2
                           + (w_cls.size + w_pred.size) * 4
                           + N * (num_classes + nc4) * 4),
    )

    scores, deltas = pl.pallas_call(
        functools.partial(_fused_body, 1.0 / float(hw)),
        out_shape=(jax.ShapeDtypeStruct((N, num_classes), jnp.float32),
                   jax.ShapeDtypeStruct((N, nc4), jnp.float32)),
        grid=grid,
        in_specs=[
            pl.BlockSpec((tn, chw), lambda i: (i, 0)),
            pl.BlockSpec((chw, C), lambda i: (0, 0)),
            pl.BlockSpec((num_classes, C), lambda i: (0, 0)),
            pl.BlockSpec((1, num_classes), lambda i: (0, 0)),
            pl.BlockSpec((nc4, C), lambda i: (0, 0)),
            pl.BlockSpec((1, nc4), lambda i: (0, 0)),
        ],
        out_specs=[
            pl.BlockSpec((tn, num_classes), lambda i: (i, 0)),
            pl.BlockSpec((tn, nc4), lambda i: (i, 0)),
        ],
        compiler_params=pltpu.CompilerParams(
            dimension_semantics=("parallel",),
            vmem_limit_bytes=48 * 1024 * 1024,
        ),
        cost_estimate=cost,
    )(xflat, sel, w_cls, bc2, w_pred, bp2)
    return scores, deltas


# speedup vs baseline: 8.0486x; 8.0486x over previous
import functools

import jax
import jax.numpy as jnp
from jax.experimental import pallas as pl
from jax.experimental.pallas import tpu as pltpu


def _fused_body(inv_hw, x_ref, wc_ref, bc_ref, wp_ref, bp_ref,
                scores_ref, deltas_ref):
    pooled = jnp.sum(x_ref[...], axis=0) * inv_hw
    dn = (((1,), (1,)), ((), ()))
    scores_ref[...] = jax.lax.dot_general(
        pooled, wc_ref[...], dn,
        preferred_element_type=jnp.float32) + bc_ref[...]
    deltas_ref[...] = jax.lax.dot_general(
        pooled, wp_ref[...], dn,
        preferred_element_type=jnp.float32) + bp_ref[...]


def kernel(x, w_cls, b_cls, w_pred, b_pred):
    num_classes = w_cls.shape[0]
    nc4 = w_pred.shape[0]

    if x.ndim == 4:
        N, C, H, W = x.shape
        hw = H * W
        xt = x.transpose(2, 3, 0, 1).reshape(hw, N, C)
    else:
        N, C = x.shape
        hw = 1
        xt = x.reshape(1, N, C)

    w_cls = jnp.asarray(w_cls, jnp.float32)
    w_pred = jnp.asarray(w_pred, jnp.float32)
    bc2 = jnp.asarray(b_cls, jnp.float32).reshape(1, num_classes)
    bp2 = jnp.asarray(b_pred, jnp.float32).reshape(1, nc4)

    if N % 128 == 0:
        tn = 128
    elif N % 8 == 0:
        tn = 8
    else:
        tn = N
    grid = (N // tn,)

    itemsize = jnp.dtype(x.dtype).itemsize
    cost = pl.CostEstimate(
        flops=int(N * C * hw + 2 * N * C * (num_classes + nc4)),
        transcendentals=0,
        bytes_accessed=int(N * C * hw * itemsize
                           + (w_cls.size + w_pred.size) * 4
                           + N * (num_classes + nc4) * 4),
    )

    scores, deltas = pl.pallas_call(
        functools.partial(_fused_body, 1.0 / float(hw)),
        out_shape=(jax.ShapeDtypeStruct((N, num_classes), jnp.float32),
                   jax.ShapeDtypeStruct((N, nc4), jnp.float32)),
        grid=grid,
        in_specs=[
            pl.BlockSpec((hw, tn, C), lambda i: (0, i, 0)),
            pl.BlockSpec((num_classes, C), lambda i: (0, 0)),
            pl.BlockSpec((1, num_classes), lambda i: (0, 0)),
            pl.BlockSpec((nc4, C), lambda i: (0, 0)),
            pl.BlockSpec((1, nc4), lambda i: (0, 0)),
        ],
        out_specs=[
            pl.BlockSpec((tn, num_classes), lambda i: (i, 0)),
            pl.BlockSpec((tn, nc4), lambda i: (i, 0)),
        ],
        compiler_params=pltpu.CompilerParams(
            dimension_semantics=("parallel",),
            vmem_limit_bytes=48 * 1024 * 1024,
        ),
        cost_estimate=cost,
    )(xt, w_cls, bc2, w_pred, bp2)
    return scores, deltas


# speedup vs baseline: 8.0965x; 1.0060x over previous
import functools

import jax
import jax.numpy as jnp
from jax.experimental import pallas as pl
from jax.experimental.pallas import tpu as pltpu


def _fused_body(inv_hw, x_ref, wc_ref, bc_ref, wp_ref, bp_ref,
                scores_ref, deltas_ref):
    pooled = jnp.sum(x_ref[...], axis=0) * inv_hw
    dn = (((1,), (1,)), ((), ()))
    scores_ref[...] = jax.lax.dot_general(
        pooled, wc_ref[...], dn,
        preferred_element_type=jnp.float32) + bc_ref[...][None, :]
    deltas_ref[...] = jax.lax.dot_general(
        pooled, wp_ref[...], dn,
        preferred_element_type=jnp.float32) + bp_ref[...][None, :]


def kernel(x, w_cls, b_cls, w_pred, b_pred):
    num_classes = w_cls.shape[0]
    nc4 = w_pred.shape[0]

    if x.ndim == 4:
        N, C, H, W = x.shape
        hw = H * W
        xt = x.transpose(2, 3, 0, 1).reshape(hw, N, C)
    else:
        N, C = x.shape
        hw = 1
        xt = x.reshape(1, N, C)


    if N % 128 == 0:
        tn = 128
    elif N % 8 == 0:
        tn = 8
    else:
        tn = N
    grid = (N // tn,)

    itemsize = jnp.dtype(x.dtype).itemsize
    cost = pl.CostEstimate(
        flops=int(N * C * hw + 2 * N * C * (num_classes + nc4)),
        transcendentals=0,
        bytes_accessed=int(N * C * hw * itemsize
                           + (w_cls.size + w_pred.size) * 4
                           + N * (num_classes + nc4) * 4),
    )

    scores, deltas = pl.pallas_call(
        functools.partial(_fused_body, 1.0 / float(hw)),
        out_shape=(jax.ShapeDtypeStruct((N, num_classes), jnp.float32),
                   jax.ShapeDtypeStruct((N, nc4), jnp.float32)),
        grid=grid,
        in_specs=[
            pl.BlockSpec((hw, tn, C), lambda i: (0, i, 0)),
            pl.BlockSpec((num_classes, C), lambda i: (0, 0)),
            pl.BlockSpec((num_classes,), lambda i: (0,)),
            pl.BlockSpec((nc4, C), lambda i: (0, 0)),
            pl.BlockSpec((nc4,), lambda i: (0,)),
        ],
        out_specs=[
            pl.BlockSpec((tn, num_classes), lambda i: (i, 0)),
            pl.BlockSpec((tn, nc4), lambda i: (i, 0)),
        ],
        compiler_params=pltpu.CompilerParams(
            dimension_semantics=("parallel",),
            vmem_limit_bytes=48 * 1024 * 1024,
        ),
        cost_estimate=cost,
    )(xt, w_cls, b_cls, w_pred, b_pred)
    return scores, deltas
